# baseline (speedup 1.0000x reference)
import functools

import jax
import jax.numpy as jnp
from jax import lax
from jax.experimental import pallas as pl
from jax.experimental.pallas import tpu as pltpu
from jax.experimental.pallas import tpu_sc as plsc

_VOCAB = 1000000
_EMB = 64
_OUT = 32
_B = 4096
_L = 200
_NTOK = _B * _L

_MB = 8192
_NBLK = -(-_VOCAB // _MB)
_NGB = 32
_G = _NGB * _MB

_NC = 2
_NS = 16
_NW = _NC * _NS
_PER_W = _NTOK // _NW
_BPW = _B // _NW
_CB = 8
_CHUNK = _CB * _L
_NCHUNK = _BPW // _CB


def _proj_body(l0, l1, l2, l3, w_ref, b_ref, out_ref):
    for j, lref in enumerate((l0, l1, l2, l3)):
        prod = lax.dot_general(
            lref[...], w_ref[...], (((0,), (0,)), ((), ())),
            preferred_element_type=jnp.float32,
        )
        out_ref[:, 32 * j:32 * (j + 1)] = prod + b_ref[...]


def _lhs_spec(j):
    def index_map(i):
        return (0, jnp.minimum(j * _NGB + i, _NBLK - 1))

    return pl.BlockSpec((_EMB, _MB), index_map)


def _project_table(emb_table, fc_w, fc_b):
    embT = emb_table.T
    packed = pl.pallas_call(
        _proj_body,
        grid=(_NGB,),
        in_specs=[_lhs_spec(j) for j in range(4)] + [
            pl.BlockSpec((_EMB, _OUT), lambda i: (0, 0)),
            pl.BlockSpec((1, _OUT), lambda i: (0, 0)),
        ],
        out_specs=pl.BlockSpec((_MB, 128), lambda i: (i, 0)),
        out_shape=jax.ShapeDtypeStruct((_G, 128), jnp.float32),
        compiler_params=pltpu.CompilerParams(vmem_limit_bytes=50 * 2**20),
    )(embT, embT, embT, embT, fc_w.T, fc_b.reshape(1, _OUT))
    return packed.reshape(4 * _G, _OUT)


_MESH = plsc.VectorSubcoreMesh(core_axis_name="c", subcore_axis_name="s")


def _make_gather(ntok, chunk):
    per_w = ntok // _NW
    nchunk = per_w // chunk

    @functools.partial(
        pl.kernel,
        mesh=_MESH,
        out_type=jax.ShapeDtypeStruct((ntok, _OUT), jnp.float32),
        scratch_types=[
            pltpu.VMEM((chunk,), jnp.int32),
            pltpu.VMEM((chunk,), jnp.int32),
            pltpu.VMEM((chunk, _OUT), jnp.float32),
            pltpu.VMEM((chunk, _OUT), jnp.float32),
            pltpu.SemaphoreType.DMA,
            pltpu.SemaphoreType.DMA,
        ],
        compiler_params=pltpu.CompilerParams(use_tc_tiling_on_sc=False),
    )
    def gather_rows(proj_hbm, idx_hbm, out_hbm, i0, i1, r0, r1, s0, s1):
        wid = lax.axis_index("s") * _NC + lax.axis_index("c")
        base = wid * per_w
        idx = (i0, i1)
        rows = (r0, r1)
        sem = (s0, s1)

        def off(j):
            return pl.multiple_of(base + j * chunk, 8)

        pltpu.sync_copy(idx_hbm.at[pl.ds(off(0), chunk)], i0)
        g = pltpu.async_copy(proj_hbm.at[i0], r0, s0)
        for j in range(nchunk):
            a = j % 2
            b = (j + 1) % 2
            if j + 1 < nchunk:
                pltpu.sync_copy(idx_hbm.at[pl.ds(off(j + 1), chunk)], idx[b])
            g.wait()
            if j + 1 < nchunk:
                g = pltpu.async_copy(proj_hbm.at[idx[b]], rows[b], sem[b])
            pltpu.sync_copy(rows[a], out_hbm.at[pl.ds(off(j), chunk)])

    return gather_rows


_gather_all = _make_gather(_NTOK, _CHUNK)


def kernel(inputs_ids, input_lens, emb_table, fc_w, fc_b):
    del input_lens
    proj = _project_table(emb_table, fc_w, fc_b)
    v = inputs_ids.reshape(_NTOK).astype(jnp.int32)
    ids_flat = lax.shift_left(v & (_G - 1), 2) | lax.shift_right_logical(v, 18)
    flat = _gather_all(proj, ids_flat)
    return flat.reshape(_B, _L, _OUT)
